# Initial kernel scaffold; baseline (speedup 1.0000x reference)
import jax
import jax.numpy as jnp
from jax.experimental import pallas as pl


def kernel(ids, num_samples, adj_info, geto_adj_info, adj_probs):
    raise NotImplementedError("write your pallas kernel here")


# speedup vs baseline: 4.6996x; 4.6996x over previous
import numpy as np
import jax
import jax.numpy as jnp
from jax import lax
from jax._src import config as jax_config
from jax.experimental import pallas as pl
from jax.experimental.pallas import tpu as pltpu
from jax.experimental.pallas import tpu_sc as plsc

N_NODES = 100000
DEG = 32
BATCH = 16384
NSAMP = 25
NSEQ = 32
TOT = BATCH * DEG

NW = 32
ROWS_W = BATCH // NW
HALF = ROWS_W // 2
IDX_CHUNK = 128

_SC_PARAMS = pltpu.CompilerParams(needs_layout_passes=False,
                                  use_tc_tiling_on_sc=False)


_ROT_A = (13, 15, 26, 6)
_ROT_B = (17, 29, 16, 24)


def _np_threefry2x32(k0, k1, x0, x1):
    with np.errstate(over="ignore"):
        k0 = np.uint32(k0); k1 = np.uint32(k1)
        k2 = np.uint32(k0 ^ k1 ^ np.uint32(0x1BD11BDA))
        x0 = np.uint32(np.uint32(x0) + k0)
        x1 = np.uint32(np.uint32(x1) + k1)
        ks = (k0, k1, k2)
        rots = (_ROT_A, _ROT_B, _ROT_A, _ROT_B, _ROT_A)
        for i in range(5):
            for r in rots[i]:
                x0 = np.uint32(x0 + x1)
                x1 = np.uint32(np.uint32(x1 << np.uint32(r))
                               | np.uint32(x1 >> np.uint32(32 - r)))
                x1 = np.uint32(x0 ^ x1)
            x0 = np.uint32(x0 + ks[(i + 1) % 3])
            x1 = np.uint32(x1 + ks[(i + 2) % 3] + np.uint32(i + 1))
    return x0, x1


_SK0, _SK1 = _np_threefry2x32(0, 0, 0, 123)
_SK2 = np.uint32(_SK0 ^ _SK1 ^ np.uint32(0x1BD11BDA))
_KSCHED = (_SK0, _SK1, _SK2)
_TINY = np.float32(np.finfo(np.float32).tiny)


def _tf_bits(mu):
    x0 = jnp.full(mu.shape, _SK0, jnp.uint32)
    x1 = mu + jnp.uint32(_SK1)
    rots = (_ROT_A, _ROT_B, _ROT_A, _ROT_B, _ROT_A)
    with np.errstate(over="ignore"):
        for i in range(5):
            for r in rots[i]:
                x0 = x0 + x1
                x1 = (x1 << jnp.uint32(r)) | (x1 >> jnp.uint32(32 - r))
                x1 = x0 ^ x1
            x0 = x0 + jnp.uint32(_KSCHED[(i + 1) % 3])
            x1 = x1 + jnp.uint32(np.uint32(_KSCHED[(i + 2) % 3]
                                           + np.uint32(i + 1)))
    return x0 ^ x1


def _compact_ids(idsp_v, idx_v):
    ev = jnp.arange(16, dtype=jnp.int32) * jnp.int32(2)
    for k in range(ROWS_W // 16):
        idx_v[pl.ds(k * 16, 16)] = plsc.load_gather(
            idsp_v, [ev + jnp.int32(32 * k)])


def _sc_gather_body(ids_hbm, geto_hbm, probs_hbm, probs_out,
                    tab_v, idsp_v, idx_v, geto_v, pr_v, sem):
    info = plsc.get_sparse_core_info()
    wid = lax.axis_index("s") * jnp.int32(info.num_cores) + lax.axis_index("c")
    base = wid * jnp.int32(ROWS_W)
    pltpu.sync_copy(probs_hbm, tab_v)
    pltpu.sync_copy(ids_hbm.at[pl.ds(base * jnp.int32(2), 2 * ROWS_W)],
                    idsp_v)
    _compact_ids(idsp_v, idx_v)
    ec0 = jnp.arange(16, dtype=jnp.int32) * jnp.int32(2)
    ec1 = ec0 + jnp.int32(32)
    for h in range(ROWS_W // HALF):
        cps = []
        for k in range(HALF // IDX_CHUNK):
            isl = idx_v.at[pl.ds(h * HALF + k * IDX_CHUNK, IDX_CHUNK)]
            cps.append(pltpu.async_copy(
                geto_hbm.at[isl],
                geto_v.at[pl.ds(k * IDX_CHUNK, IDX_CHUNK)], sem))
        for cp in cps:
            cp.wait()

        def grp_body(_, g):
            for q in range(4):
                i = g * jnp.int32(4) + jnp.int32(q)
                iv = jnp.full((16,), i, jnp.int32)
                for jj, ec in ((0, ec0), (1, ec1)):
                    nvec = plsc.load_gather(geto_v, [iv, ec])
                    pr_v[g, pl.ds(32 * q + 16 * jj, 16)] = plsc.load_gather(
                        tab_v, [nvec])
            return g + jnp.int32(1)

        lax.fori_loop(jnp.int32(0), jnp.int32(HALF // 4), grp_body,
                      jnp.int32(0))
        pltpu.sync_copy(
            pr_v,
            probs_out.at[pl.ds(wid * jnp.int32(128) + jnp.int32(h * 64), 64)])


def _sc_gather(idsp, getop, probs):
    fn = pl.kernel(
        _sc_gather_body,
        out_type=jax.ShapeDtypeStruct((TOT // 128, 128), jnp.float32),
        compiler_params=_SC_PARAMS,
        mesh=plsc.VectorSubcoreMesh(core_axis_name="c", subcore_axis_name="s"),
        scratch_types=[
            pltpu.VMEM((N_NODES,), jnp.float32),
            pltpu.VMEM((2 * ROWS_W,), jnp.int32),
            pltpu.VMEM((ROWS_W,), jnp.int32),
            pltpu.VMEM((HALF, 2 * DEG), jnp.int32),
            pltpu.VMEM((64, 128), jnp.float32),
            pltpu.SemaphoreType.DMA,
        ],
    )
    return fn(idsp, getop, probs)


_CH = 64
_NCHUNK = TOT // (_CH * 128)
_FROWS = TOT // 128


def _tc_sample_body(ns_ref, probs_ref, out_ref, cols_ref, logits_ref):
    s = pl.program_id(0)

    @pl.when(s == 0)
    def _():
        logits_ref[:] = jnp.log(probs_ref[:])

    ns = ns_ref[0]
    lo = jnp.clip(ns - jnp.int32(NSAMP), jnp.int32(0), jnp.int32(NSEQ - 1))
    hi = jnp.clip(ns - jnp.int32(1), jnp.int32(0), jnp.int32(NSEQ - 1))
    out_ref[s] = jnp.int32(0)

    @pl.when((s >= lo) & (s <= hi))
    def _():
        it = (lax.broadcasted_iota(jnp.int32, (_CH, 128), 0)
              * jnp.int32(128)
              + lax.broadcasted_iota(jnp.int32, (_CH, 128), 1))

        def chunk(_, carry):
            ci, accv, accj = carry
            jvec = ci * jnp.int32(_CH * 128) + it
            mu = (s * jnp.int32(TOT) + jvec).astype(jnp.uint32)
            bits = _tf_bits(mu)
            fb = (bits >> jnp.uint32(9)) | jnp.uint32(0x3F800000)
            fl = lax.bitcast_convert_type(fb, jnp.float32) - np.float32(1.0)
            u = jnp.maximum(_TINY, fl + _TINY)
            g = -jnp.log(-jnp.log(u))
            v = g + logits_ref[pl.ds(ci * jnp.int32(_CH), _CH), :]
            pred = v > accv
            return (ci + jnp.int32(1),
                    jnp.where(pred, v, accv),
                    jnp.where(pred, jvec, accj))

        init = (jnp.int32(0),
                jnp.full((_CH, 128), -np.inf, jnp.float32),
                jnp.zeros((_CH, 128), jnp.int32))
        _, accv, accj = lax.fori_loop(jnp.int32(0), jnp.int32(_NCHUNK),
                                      chunk, init)
        vmax = jnp.max(accv)
        bidx = jnp.min(jnp.where(accv == vmax, accj, jnp.int32(TOT)))
        out_ref[s] = jnp.minimum(bidx, jnp.int32(NSEQ - 1))

    @pl.when(s == jnp.int32(NSEQ - 1))
    def _():
        for t in range(NSAMP):
            st = jnp.clip(jnp.int32(t) + ns - jnp.int32(NSAMP),
                          jnp.int32(0), jnp.int32(NSEQ - 1))
            cols_ref[t] = out_ref[st]
        for t in range(NSAMP, NSEQ):
            cols_ref[t] = jnp.int32(0)


def _tc_sample(ns32, probs_flat):
    return pl.pallas_call(
        _tc_sample_body,
        grid=(NSEQ,),
        in_specs=[
            pl.BlockSpec(memory_space=pltpu.SMEM),
            pl.BlockSpec((_FROWS, 128), lambda s: (0, 0)),
        ],
        out_specs=[pl.BlockSpec(memory_space=pltpu.SMEM),
                   pl.BlockSpec(memory_space=pltpu.SMEM)],
        out_shape=(jax.ShapeDtypeStruct((NSEQ,), jnp.int32),
                   jax.ShapeDtypeStruct((NSEQ,), jnp.int32)),
        scratch_shapes=[pltpu.VMEM((_FROWS, 128), jnp.float32)],
    )(ns32, probs_flat)


def _sc_select_body(ids_hbm, adj_hbm, geto_hbm, cols_hbm,
                    oa_hbm, og_hbm,
                    idsp_v, idx_v, cols_v, adj_v, geto_v, oa_v, og_v, sem):
    info = plsc.get_sparse_core_info()
    wid = lax.axis_index("s") * jnp.int32(info.num_cores) + lax.axis_index("c")
    base = wid * jnp.int32(ROWS_W)
    pltpu.sync_copy(cols_hbm, cols_v)
    pltpu.sync_copy(ids_hbm.at[pl.ds(base * jnp.int32(2), 2 * ROWS_W)],
                    idsp_v)
    _compact_ids(idsp_v, idx_v)
    lane = jnp.arange(16, dtype=jnp.int32)
    pairsel = lane >> jnp.int32(1)
    par = lane & jnp.int32(1)
    widx = []
    for k in range(4):
        q = plsc.load_gather(cols_v, [pairsel + jnp.int32(8 * k)])
        widx.append(q * jnp.int32(2) + par)
    for h in range(ROWS_W // HALF):
        cps = []
        for k in range(HALF // IDX_CHUNK):
            isl = idx_v.at[pl.ds(h * HALF + k * IDX_CHUNK, IDX_CHUNK)]
            dsl = pl.ds(k * IDX_CHUNK, IDX_CHUNK)
            cps.append(pltpu.async_copy(adj_hbm.at[isl], adj_v.at[dsl], sem))
            cps.append(pltpu.async_copy(geto_hbm.at[isl], geto_v.at[dsl],
                                        sem))
        for cp in cps:
            cp.wait()

        def row_body(_, i):
            iv = jnp.full((16,), i, jnp.int32)
            for k in range(4):
                oa_v[i, pl.ds(16 * k, 16)] = plsc.load_gather(
                    adj_v, [iv, widx[k]])
                og_v[i, pl.ds(16 * k, 16)] = plsc.load_gather(
                    geto_v, [iv, widx[k]])
            return i + jnp.int32(1)

        lax.fori_loop(jnp.int32(0), jnp.int32(HALF), row_body, jnp.int32(0))
        hb = base + jnp.int32(h * HALF)
        pltpu.sync_copy(oa_v, oa_hbm.at[pl.ds(hb, HALF)])
        pltpu.sync_copy(og_v, og_hbm.at[pl.ds(hb, HALF)])


def _sc_select(idsp, adjp, getop, cols32):
    fn = pl.kernel(
        _sc_select_body,
        out_type=(jax.ShapeDtypeStruct((BATCH, 2 * DEG), jnp.int32),
                  jax.ShapeDtypeStruct((BATCH, 2 * DEG), jnp.int32)),
        compiler_params=_SC_PARAMS,
        mesh=plsc.VectorSubcoreMesh(core_axis_name="c", subcore_axis_name="s"),
        scratch_types=[
            pltpu.VMEM((2 * ROWS_W,), jnp.int32),
            pltpu.VMEM((ROWS_W,), jnp.int32),
            pltpu.VMEM((NSEQ,), jnp.int32),
            pltpu.VMEM((HALF, 2 * DEG), jnp.int32),
            pltpu.VMEM((HALF, 2 * DEG), jnp.int32),
            pltpu.VMEM((HALF, 2 * DEG), jnp.int32),
            pltpu.VMEM((HALF, 2 * DEG), jnp.int32),
            pltpu.SemaphoreType.DMA,
        ],
    )
    return fn(idsp, adjp, getop, cols32)


def kernel(ids, num_samples, adj_info, geto_adj_info, adj_probs):
    with jax_config.enable_x64(False):
        idsp = lax.bitcast_convert_type(ids, jnp.int32).reshape(2 * BATCH)
        adjp = lax.bitcast_convert_type(adj_info, jnp.int32).reshape(
            N_NODES, 2 * DEG)
        getop = lax.bitcast_convert_type(geto_adj_info, jnp.int32).reshape(
            N_NODES, 2 * DEG)
        ns32 = jnp.asarray(num_samples, jnp.int32).reshape(1)

        probs_flat = _sc_gather(idsp, getop, adj_probs)
        _, cols32 = _tc_sample(ns32, probs_flat)
        oa_p, og_p = _sc_select(idsp, adjp, getop, cols32)
        oa_pairs = oa_p.reshape(BATCH, DEG, 2)
        og_pairs = og_p.reshape(BATCH, DEG, 2)
    oa = lax.bitcast_convert_type(oa_pairs, jnp.int64)[:, :NSAMP]
    og = lax.bitcast_convert_type(og_pairs, jnp.int64)[:, :NSAMP]
    return (oa, og)


# speedup vs baseline: 4.7095x; 1.0021x over previous
import numpy as np
import jax
import jax.numpy as jnp
from jax import lax
from jax._src import config as jax_config
from jax.experimental import pallas as pl
from jax.experimental.pallas import tpu as pltpu
from jax.experimental.pallas import tpu_sc as plsc

N_NODES = 100000
DEG = 32
BATCH = 16384
NSAMP = 25
NSEQ = 32
TOT = BATCH * DEG

NW = 32
ROWS_W = BATCH // NW
HALF = ROWS_W // 2
IDX_CHUNK = 128

_SC_PARAMS = pltpu.CompilerParams(needs_layout_passes=False,
                                  use_tc_tiling_on_sc=False)


_ROT_A = (13, 15, 26, 6)
_ROT_B = (17, 29, 16, 24)


def _np_threefry2x32(k0, k1, x0, x1):
    with np.errstate(over="ignore"):
        k0 = np.uint32(k0); k1 = np.uint32(k1)
        k2 = np.uint32(k0 ^ k1 ^ np.uint32(0x1BD11BDA))
        x0 = np.uint32(np.uint32(x0) + k0)
        x1 = np.uint32(np.uint32(x1) + k1)
        ks = (k0, k1, k2)
        rots = (_ROT_A, _ROT_B, _ROT_A, _ROT_B, _ROT_A)
        for i in range(5):
            for r in rots[i]:
                x0 = np.uint32(x0 + x1)
                x1 = np.uint32(np.uint32(x1 << np.uint32(r))
                               | np.uint32(x1 >> np.uint32(32 - r)))
                x1 = np.uint32(x0 ^ x1)
            x0 = np.uint32(x0 + ks[(i + 1) % 3])
            x1 = np.uint32(x1 + ks[(i + 2) % 3] + np.uint32(i + 1))
    return x0, x1


_SK0, _SK1 = _np_threefry2x32(0, 0, 0, 123)
_SK2 = np.uint32(_SK0 ^ _SK1 ^ np.uint32(0x1BD11BDA))
_KSCHED = (_SK0, _SK1, _SK2)
_TINY = np.float32(np.finfo(np.float32).tiny)


def _tf_bits(mu):
    x0 = jnp.full(mu.shape, _SK0, jnp.uint32)
    x1 = mu + jnp.uint32(_SK1)
    rots = (_ROT_A, _ROT_B, _ROT_A, _ROT_B, _ROT_A)
    with np.errstate(over="ignore"):
        for i in range(5):
            for r in rots[i]:
                x0 = x0 + x1
                x1 = (x1 << jnp.uint32(r)) | (x1 >> jnp.uint32(32 - r))
                x1 = x0 ^ x1
            x0 = x0 + jnp.uint32(_KSCHED[(i + 1) % 3])
            x1 = x1 + jnp.uint32(np.uint32(_KSCHED[(i + 2) % 3]
                                           + np.uint32(i + 1)))
    return x0 ^ x1


def _compact_ids(idsp_v, idx_v):
    ev = jnp.arange(16, dtype=jnp.int32) * jnp.int32(2)
    for k in range(ROWS_W // 16):
        idx_v[pl.ds(k * 16, 16)] = plsc.load_gather(
            idsp_v, [ev + jnp.int32(32 * k)])


def _sc_gather_body(ids_hbm, geto_hbm, probs_hbm, probs_out,
                    tab_v, idsp_v, idx_v, geto_v, pr_v, sem):
    info = plsc.get_sparse_core_info()
    wid = lax.axis_index("s") * jnp.int32(info.num_cores) + lax.axis_index("c")
    base = wid * jnp.int32(ROWS_W)
    pltpu.sync_copy(probs_hbm, tab_v)
    pltpu.sync_copy(ids_hbm.at[pl.ds(base * jnp.int32(2), 2 * ROWS_W)],
                    idsp_v)
    _compact_ids(idsp_v, idx_v)
    ec0 = jnp.arange(16, dtype=jnp.int32) * jnp.int32(2)
    ec1 = ec0 + jnp.int32(32)
    for h in range(ROWS_W // HALF):
        cps = []
        for k in range(HALF // IDX_CHUNK):
            isl = idx_v.at[pl.ds(h * HALF + k * IDX_CHUNK, IDX_CHUNK)]
            cps.append(pltpu.async_copy(
                geto_hbm.at[isl],
                geto_v.at[pl.ds(k * IDX_CHUNK, IDX_CHUNK)], sem))
        for cp in cps:
            cp.wait()

        def grp_body(_, g):
            for q in range(4):
                i = g * jnp.int32(4) + jnp.int32(q)
                iv = jnp.full((16,), i, jnp.int32)
                for jj, ec in ((0, ec0), (1, ec1)):
                    nvec = plsc.load_gather(geto_v, [iv, ec])
                    pr_v[g, pl.ds(32 * q + 16 * jj, 16)] = plsc.load_gather(
                        tab_v, [nvec])
            return g + jnp.int32(1)

        lax.fori_loop(jnp.int32(0), jnp.int32(HALF // 4), grp_body,
                      jnp.int32(0))
        pltpu.sync_copy(
            pr_v,
            probs_out.at[pl.ds(wid * jnp.int32(128) + jnp.int32(h * 64), 64)])


def _sc_gather(idsp, getop, probs):
    fn = pl.kernel(
        _sc_gather_body,
        out_type=jax.ShapeDtypeStruct((TOT // 128, 128), jnp.float32),
        compiler_params=_SC_PARAMS,
        mesh=plsc.VectorSubcoreMesh(core_axis_name="c", subcore_axis_name="s"),
        scratch_types=[
            pltpu.VMEM((N_NODES,), jnp.float32),
            pltpu.VMEM((2 * ROWS_W,), jnp.int32),
            pltpu.VMEM((ROWS_W,), jnp.int32),
            pltpu.VMEM((HALF, 2 * DEG), jnp.int32),
            pltpu.VMEM((64, 128), jnp.float32),
            pltpu.SemaphoreType.DMA,
        ],
    )
    return fn(idsp, getop, probs)


_CH = 64
_NCHUNK = TOT // (_CH * 128)
_FROWS = TOT // 128


def _tc_sample_body(ns_ref, probs_ref, out_ref, cols_ref, logits_ref):
    s = pl.program_id(0)

    @pl.when(s == 0)
    def _():
        logits_ref[:] = jnp.log(probs_ref[:])

    ns = ns_ref[0]
    lo = jnp.clip(ns - jnp.int32(NSAMP), jnp.int32(0), jnp.int32(NSEQ - 1))
    hi = jnp.clip(ns - jnp.int32(1), jnp.int32(0), jnp.int32(NSEQ - 1))
    out_ref[s] = jnp.int32(0)

    @pl.when((s >= lo) & (s <= hi))
    def _():
        it = (lax.broadcasted_iota(jnp.int32, (_CH, 128), 0)
              * jnp.int32(128)
              + lax.broadcasted_iota(jnp.int32, (_CH, 128), 1))

        def chunk(_, carry):
            ci, accv, accj = carry
            jvec = ci * jnp.int32(_CH * 128) + it
            mu = (s * jnp.int32(TOT) + jvec).astype(jnp.uint32)
            bits = _tf_bits(mu)
            fb = (bits >> jnp.uint32(9)) | jnp.uint32(0x3F800000)
            fl = lax.bitcast_convert_type(fb, jnp.float32) - np.float32(1.0)
            u = fl + _TINY
            v = (logits_ref[pl.ds(ci * jnp.int32(_CH), _CH), :]
                 - jnp.log(-jnp.log(u)))
            pred = v > accv
            return (ci + jnp.int32(1),
                    jnp.where(pred, v, accv),
                    jnp.where(pred, jvec, accj))

        init = (jnp.int32(0),
                jnp.full((_CH, 128), -np.inf, jnp.float32),
                jnp.zeros((_CH, 128), jnp.int32))
        _, accv, accj = lax.fori_loop(jnp.int32(0), jnp.int32(_NCHUNK),
                                      chunk, init)
        vmax = jnp.max(accv)
        bidx = jnp.min(jnp.where(accv == vmax, accj, jnp.int32(TOT)))
        out_ref[s] = jnp.minimum(bidx, jnp.int32(NSEQ - 1))

    @pl.when(s == jnp.int32(NSEQ - 1))
    def _():
        for t in range(NSAMP):
            st = jnp.clip(jnp.int32(t) + ns - jnp.int32(NSAMP),
                          jnp.int32(0), jnp.int32(NSEQ - 1))
            cols_ref[t] = out_ref[st]
        for t in range(NSAMP, NSEQ):
            cols_ref[t] = jnp.int32(0)


def _tc_sample(ns32, probs_flat):
    return pl.pallas_call(
        _tc_sample_body,
        grid=(NSEQ,),
        in_specs=[
            pl.BlockSpec(memory_space=pltpu.SMEM),
            pl.BlockSpec((_FROWS, 128), lambda s: (0, 0)),
        ],
        out_specs=[pl.BlockSpec(memory_space=pltpu.SMEM),
                   pl.BlockSpec(memory_space=pltpu.SMEM)],
        out_shape=(jax.ShapeDtypeStruct((NSEQ,), jnp.int32),
                   jax.ShapeDtypeStruct((NSEQ,), jnp.int32)),
        scratch_shapes=[pltpu.VMEM((_FROWS, 128), jnp.float32)],
    )(ns32, probs_flat)


def _sc_select_body(ids_hbm, adj_hbm, geto_hbm, cols_hbm,
                    oa_hbm, og_hbm,
                    idsp_v, idx_v, cols_v, adj_v, geto_v, oa_v, og_v, sem):
    info = plsc.get_sparse_core_info()
    wid = lax.axis_index("s") * jnp.int32(info.num_cores) + lax.axis_index("c")
    base = wid * jnp.int32(ROWS_W)
    pltpu.sync_copy(cols_hbm, cols_v)
    pltpu.sync_copy(ids_hbm.at[pl.ds(base * jnp.int32(2), 2 * ROWS_W)],
                    idsp_v)
    _compact_ids(idsp_v, idx_v)
    lane = jnp.arange(16, dtype=jnp.int32)
    pairsel = lane >> jnp.int32(1)
    par = lane & jnp.int32(1)
    widx = []
    for k in range(4):
        q = plsc.load_gather(cols_v, [pairsel + jnp.int32(8 * k)])
        widx.append(q * jnp.int32(2) + par)
    for h in range(ROWS_W // HALF):
        cps = []
        for k in range(HALF // IDX_CHUNK):
            isl = idx_v.at[pl.ds(h * HALF + k * IDX_CHUNK, IDX_CHUNK)]
            dsl = pl.ds(k * IDX_CHUNK, IDX_CHUNK)
            cps.append(pltpu.async_copy(adj_hbm.at[isl], adj_v.at[dsl], sem))
            cps.append(pltpu.async_copy(geto_hbm.at[isl], geto_v.at[dsl],
                                        sem))
        for cp in cps:
            cp.wait()

        def row_body(_, i):
            iv = jnp.full((16,), i, jnp.int32)
            for k in range(4):
                oa_v[i, pl.ds(16 * k, 16)] = plsc.load_gather(
                    adj_v, [iv, widx[k]])
                og_v[i, pl.ds(16 * k, 16)] = plsc.load_gather(
                    geto_v, [iv, widx[k]])
            return i + jnp.int32(1)

        lax.fori_loop(jnp.int32(0), jnp.int32(HALF), row_body, jnp.int32(0))
        hb = base + jnp.int32(h * HALF)
        pltpu.sync_copy(oa_v, oa_hbm.at[pl.ds(hb, HALF)])
        pltpu.sync_copy(og_v, og_hbm.at[pl.ds(hb, HALF)])


def _sc_select(idsp, adjp, getop, cols32):
    fn = pl.kernel(
        _sc_select_body,
        out_type=(jax.ShapeDtypeStruct((BATCH, 2 * DEG), jnp.int32),
                  jax.ShapeDtypeStruct((BATCH, 2 * DEG), jnp.int32)),
        compiler_params=_SC_PARAMS,
        mesh=plsc.VectorSubcoreMesh(core_axis_name="c", subcore_axis_name="s"),
        scratch_types=[
            pltpu.VMEM((2 * ROWS_W,), jnp.int32),
            pltpu.VMEM((ROWS_W,), jnp.int32),
            pltpu.VMEM((NSEQ,), jnp.int32),
            pltpu.VMEM((HALF, 2 * DEG), jnp.int32),
            pltpu.VMEM((HALF, 2 * DEG), jnp.int32),
            pltpu.VMEM((HALF, 2 * DEG), jnp.int32),
            pltpu.VMEM((HALF, 2 * DEG), jnp.int32),
            pltpu.SemaphoreType.DMA,
        ],
    )
    return fn(idsp, adjp, getop, cols32)


def kernel(ids, num_samples, adj_info, geto_adj_info, adj_probs):
    with jax_config.enable_x64(False):
        idsp = lax.bitcast_convert_type(ids, jnp.int32).reshape(2 * BATCH)
        adjp = lax.bitcast_convert_type(adj_info, jnp.int32).reshape(
            N_NODES, 2 * DEG)
        getop = lax.bitcast_convert_type(geto_adj_info, jnp.int32).reshape(
            N_NODES, 2 * DEG)
        ns32 = jnp.asarray(num_samples, jnp.int32).reshape(1)

        probs_flat = _sc_gather(idsp, getop, adj_probs)
        _, cols32 = _tc_sample(ns32, probs_flat)
        oa_p, og_p = _sc_select(idsp, adjp, getop, cols32)
        oa_pairs = oa_p.reshape(BATCH, DEG, 2)
        og_pairs = og_p.reshape(BATCH, DEG, 2)
    oa = lax.bitcast_convert_type(oa_pairs, jnp.int64)[:, :NSAMP]
    og = lax.bitcast_convert_type(og_pairs, jnp.int64)[:, :NSAMP]
    return (oa, og)
